# Initial kernel scaffold; baseline (speedup 1.0000x reference)
import jax
import jax.numpy as jnp
from jax.experimental import pallas as pl


def kernel(x_u, x_v, x_y, edge_index_u, edge_index_v, edge_index_y, u_ew1, u_eb1, u_ew2, u_eb2, u_nw1, u_nb1, u_nw2, u_nb2, v_ew1, v_eb1, v_ew2, v_eb2, v_nw1, v_nb1, v_nw2, v_nb2, y_ew1, y_eb1, y_ew2, y_eb2, y_nw1, y_nb1, y_nw2, y_nb2):
    raise NotImplementedError("write your pallas kernel here")


# speedup vs baseline: 1.0349x; 1.0349x over previous
import functools

import jax
import jax.numpy as jnp
import numpy as np
from jax.experimental import pallas as pl
from jax.experimental.pallas import tpu as pltpu

C = 5
EDGE = 16
NODE = 64


def _proj_body(x_ref, wi_ref, wj_ref, bi_ref, ai_ref, aj_ref):
    x = x_ref[...]
    d = wi_ref.shape[1]
    for c in range(C):
        xc = x[:, c * d:(c + 1) * d]
        ai = jnp.dot(xc, wi_ref[c], preferred_element_type=jnp.float32)
        aj = jnp.dot(xc, wj_ref[c], preferred_element_type=jnp.float32)
        ai_ref[:, c * EDGE:(c + 1) * EDGE] = ai + bi_ref[c][None, :]
        aj_ref[:, c * EDGE:(c + 1) * EDGE] = aj


def _edge_w_body(gd_ref, gs_ref, b2_ref, eb2_ref, w_ref):
    t = jnp.tanh(gd_ref[...] + gs_ref[...])
    logits = jnp.dot(t, b2_ref[...], preferred_element_type=jnp.float32)
    logits = logits + eb2_ref[...]
    m = jnp.max(logits, axis=1, keepdims=True)
    e = jnp.exp(logits - m)
    w_ref[...] = e / jnp.sum(e, axis=1, keepdims=True)


def _node_body(x_ref, aggr_ref, w1_ref, b1_ref, w2_ref, b2_ref, out_ref):
    x = x_ref[...]
    a = aggr_ref[...]
    d = w1_ref.shape[1] // 2
    for c in range(C):
        xc = x[:, c * d:(c + 1) * d]
        ac = a[:, c * d:(c + 1) * d]
        h = jnp.dot(xc, w1_ref[c][:d], preferred_element_type=jnp.float32)
        h = h + jnp.dot(ac, w1_ref[c][d:], preferred_element_type=jnp.float32)
        h = jnp.tanh(h + b1_ref[c][None, :])
        o = jnp.dot(h, w2_ref[c], preferred_element_type=jnp.float32)
        out_ref[:, c * NODE:(c + 1) * NODE] = jnp.tanh(o + b2_ref[c][None, :])


def _plane(x, edge_index, ew1, eb1, ew2, eb2, nw1, nb1, nw2, nb2):
    n, _, d = x.shape
    e = edge_index.shape[1]
    src = edge_index[0]
    dst = edge_index[1]
    xf = x.reshape(n, C * d)

    wi = jnp.transpose(ew1[:, :, :d], (0, 2, 1))
    wj = jnp.transpose(ew1[:, :, d:], (0, 2, 1))
    bn = 1000
    ai, aj = pl.pallas_call(
        _proj_body,
        grid=(n // bn,),
        in_specs=[
            pl.BlockSpec((bn, C * d), lambda i: (i, 0)),
            pl.BlockSpec((C, d, EDGE), lambda i: (0, 0, 0)),
            pl.BlockSpec((C, d, EDGE), lambda i: (0, 0, 0)),
            pl.BlockSpec((C, EDGE), lambda i: (0, 0)),
        ],
        out_specs=[
            pl.BlockSpec((bn, C * EDGE), lambda i: (i, 0)),
            pl.BlockSpec((bn, C * EDGE), lambda i: (i, 0)),
        ],
        out_shape=[
            jax.ShapeDtypeStruct((n, C * EDGE), jnp.float32),
            jax.ShapeDtypeStruct((n, C * EDGE), jnp.float32),
        ],
    )(xf, wi, wj, eb1)

    gd = ai[dst]
    gs = aj[src]
    b2 = jnp.zeros((C * EDGE, C), jnp.float32)
    for c in range(C):
        b2 = b2.at[c * EDGE:(c + 1) * EDGE, c].set(ew2[c, 0, :])
    be = 2000
    w = pl.pallas_call(
        _edge_w_body,
        grid=(e // be,),
        in_specs=[
            pl.BlockSpec((be, C * EDGE), lambda i: (i, 0)),
            pl.BlockSpec((be, C * EDGE), lambda i: (i, 0)),
            pl.BlockSpec((C * EDGE, C), lambda i: (0, 0)),
            pl.BlockSpec((1, C), lambda i: (0, 0)),
        ],
        out_specs=pl.BlockSpec((be, C), lambda i: (i, 0)),
        out_shape=jax.ShapeDtypeStruct((e, C), jnp.float32),
    )(gd, gs, b2, eb2.reshape(1, C))

    msg = w[:, :, None] * x[src]
    aggr = jax.ops.segment_sum(msg, dst, num_segments=n)

    w1 = jnp.transpose(nw1, (0, 2, 1))
    w2 = jnp.transpose(nw2, (0, 2, 1))
    out = pl.pallas_call(
        _node_body,
        grid=(n // bn,),
        in_specs=[
            pl.BlockSpec((bn, C * d), lambda i: (i, 0)),
            pl.BlockSpec((bn, C * d), lambda i: (i, 0)),
            pl.BlockSpec((C, 2 * d, NODE), lambda i: (0, 0, 0)),
            pl.BlockSpec((C, NODE), lambda i: (0, 0)),
            pl.BlockSpec((C, NODE, NODE), lambda i: (0, 0, 0)),
            pl.BlockSpec((C, NODE), lambda i: (0, 0)),
        ],
        out_specs=pl.BlockSpec((bn, C * NODE), lambda i: (i, 0)),
        out_shape=jax.ShapeDtypeStruct((n, C * NODE), jnp.float32),
    )(xf, aggr.reshape(n, C * d), w1, nb1, w2, nb2)
    return out.reshape(n, C, NODE)


def kernel(x_u, x_v, x_y, edge_index_u, edge_index_v, edge_index_y,
           u_ew1, u_eb1, u_ew2, u_eb2, u_nw1, u_nb1, u_nw2, u_nb2,
           v_ew1, v_eb1, v_ew2, v_eb2, v_nw1, v_nb1, v_nw2, v_nb2,
           y_ew1, y_eb1, y_ew2, y_eb2, y_nw1, y_nb1, y_nw2, y_nb2):
    out_u = _plane(x_u, edge_index_u, u_ew1, u_eb1, u_ew2, u_eb2,
                   u_nw1, u_nb1, u_nw2, u_nb2)
    out_v = _plane(x_v, edge_index_v, v_ew1, v_eb1, v_ew2, v_eb2,
                   v_nw1, v_nb1, v_nw2, v_nb2)
    out_y = _plane(x_y, edge_index_y, y_ew1, y_eb1, y_ew2, y_eb2,
                   y_nw1, y_nb1, y_nw2, y_nb2)
    return (out_u, out_v, out_y)


# speedup vs baseline: 11.5751x; 11.1847x over previous
import functools

import jax
import jax.numpy as jnp
from jax import lax
from jax.experimental import pallas as pl
from jax.experimental.pallas import tpu as pltpu
from jax.experimental.pallas import tpu_sc as plsc

C = 5
EDGE = 16
NODE = 64
F = C * EDGE
DP = 80
K = 128
NTILES = 32


def _proj_body(x_ref, wi_ref, wj_ref, bi_ref, ai_ref, aj_ref):
    x = x_ref[...]
    d = wi_ref.shape[1]
    for c in range(C):
        xc = x[:, c * d:(c + 1) * d]
        ai = jnp.dot(xc, wi_ref[c], preferred_element_type=jnp.float32)
        aj = jnp.dot(xc, wj_ref[c], preferred_element_type=jnp.float32)
        ai_ref[:, c * EDGE:(c + 1) * EDGE] = 2.0 * (ai + bi_ref[c][None, :])
        aj_ref[:, c * EDGE:(c + 1) * EDGE] = 2.0 * aj


def _node_body(x_ref, a0_ref, a1_ref, w1_ref, b1_ref, w2_ref, b2_ref,
               out_ref):
    c = pl.program_id(0)
    d = w1_ref.shape[1] // 2
    xc = x_ref[0, :, :d]
    ac = a0_ref[0, 0, :, :d] + a1_ref[0, 0, :, :d]
    h = jnp.dot(xc, w1_ref[0, :d], preferred_element_type=jnp.float32)
    h = h + jnp.dot(ac, w1_ref[0, d:], preferred_element_type=jnp.float32)
    h = jnp.tanh(h + b1_ref[c][None, :])
    o = jnp.dot(h, w2_ref[0], preferred_element_type=jnp.float32)
    out_ref[0] = jnp.tanh(o + b2_ref[c][None, :])


def _sc_body(ai2, aj2, xt2, eidx, ew2s, eb2s, zeros_hbm, aggr_out,
             srcidx, dstidx, wbuf, ga, gx, msg, srcc, ew2v, eb2v,
             aggr_sh, sem):
    n = zeros_hbm.shape[0] * 16
    e = eidx.shape[1]
    nch_tot = e // K
    nper = n // 16

    ci = lax.axis_index("c")
    si = lax.axis_index("s")
    wid = si * 2 + ci
    nch = nch_tot // NTILES + jnp.where(wid < nch_tot % NTILES, 1, 0)

    pltpu.sync_copy(ew2s, ew2v)
    pltpu.sync_copy(eb2s, eb2v)
    iota = lax.iota(jnp.int32, 16)

    def pass_a(l, carry):
        g = wid + l * NTILES
        base = g * K
        pltpu.sync_copy(eidx.at[0, pl.ds(base, K)], srcidx.at[l])
        pltpu.sync_copy(eidx.at[1, pl.ds(base, K)], dstidx.at[l])
        pltpu.async_copy(ai2.at[dstidx.at[l]], ga, sem).wait()
        pltpu.async_copy(aj2.at[srcidx.at[l]], ga, sem, add=True).wait()

        def grp_body(grp, cc):
            rows = iota + grp * 16
            accs = []
            for c in range(C):
                def fbody(fb, acc, c=c, rows=rows):
                    for ff in range(4):
                        row = c * 16 + fb * 4 + ff
                        colv = jnp.full((16,), row, jnp.int32)
                        z = plsc.load_gather(ga, [rows, colv])
                        ez = jnp.exp(z)
                        t = (ez - 1.0) / (ez + 1.0)
                        wv = plsc.load_gather(ew2v, [colv, iota])
                        acc = acc + wv * t
                    return acc
                accs.append(lax.fori_loop(0, 4, fbody, eb2v[c, :]))
            m = accs[0]
            for c in range(1, C):
                m = jnp.maximum(m, accs[c])
            es = [jnp.exp(a - m) for a in accs]
            tot = es[0]
            for c in range(1, C):
                tot = tot + es[c]
            r = 1.0 / tot
            for c in range(C):
                wbuf[c, l, pl.ds(grp * 16, 16)] = es[c] * r
            return cc

        lax.fori_loop(0, 8, grp_body, 0)
        return carry

    lax.fori_loop(0, nch, pass_a, 0)

    def cls_body(cls, carry):
        pltpu.sync_copy(zeros_hbm, aggr_sh.at[pl.ds(si * nper, nper)])
        plsc.subcore_barrier()

        def pass_b(l, cc):
            def jb(j, c2):
                srcc[pl.ds(j * 16, 16)] = (
                    srcidx[l, pl.ds(j * 16, 16)] + cls * n)
                return c2
            lax.fori_loop(0, 8, jb, 0)
            pltpu.async_copy(xt2.at[srcc], gx, sem).wait()

            def grp_body(grp, c3):
                rows = iota + grp * 16
                wv = wbuf[cls, l, pl.ds(grp * 16, 16)]
                def fb2(fb, c4, rows=rows, wv=wv):
                    for ff in range(17):
                        colv = jnp.full((16,), fb * 17 + ff, jnp.int32)
                        xc = plsc.load_gather(gx, [rows, colv])
                        plsc.store_scatter(msg, [rows, colv], xc * wv)
                    return c4
                lax.fori_loop(0, 4, fb2, 0)
                return c3
            lax.fori_loop(0, 8, grp_body, 0)
            pltpu.sync_copy(msg, aggr_sh.at[dstidx.at[l]], add=True)
            return cc

        lax.fori_loop(0, nch, pass_b, 0)
        plsc.subcore_barrier()
        pltpu.sync_copy(aggr_sh.at[pl.ds(si * nper, nper)],
                        aggr_out.at[ci, cls, si])
        plsc.subcore_barrier()
        return carry

    lax.fori_loop(0, C, cls_body, 0)


def _make_sc_call(n, e, nch_max):
    return pl.kernel(
        _sc_body,
        out_type=jax.ShapeDtypeStruct((2, C, 16, n // 16, DP), jnp.float32),
        mesh=plsc.VectorSubcoreMesh(core_axis_name="c", subcore_axis_name="s"),
        compiler_params=pltpu.CompilerParams(
            needs_layout_passes=False, use_tc_tiling_on_sc=False),
        scratch_types=[
            pltpu.VMEM((nch_max, K), jnp.int32),
            pltpu.VMEM((nch_max, K), jnp.int32),
            pltpu.VMEM((C, nch_max, K), jnp.float32),
            pltpu.VMEM((K, F), jnp.float32),
            pltpu.VMEM((K, DP), jnp.float32),
            pltpu.VMEM((K, DP), jnp.float32),
            pltpu.VMEM((K,), jnp.int32),
            pltpu.VMEM((F, 16), jnp.float32),
            pltpu.VMEM((C, 16), jnp.float32),
            pltpu.VMEM_SHARED((n, DP), jnp.float32),
            pltpu.SemaphoreType.DMA,
        ],
    )


def _plane(x, edge_index, ew1, eb1, ew2, eb2, nw1, nb1, nw2, nb2):
    n, _, d = x.shape
    e = edge_index.shape[1]
    xf = x.reshape(n, C * d)

    wi = jnp.transpose(ew1[:, :, :d], (0, 2, 1))
    wj = jnp.transpose(ew1[:, :, d:], (0, 2, 1))
    bn = 1000
    ai2, aj2 = pl.pallas_call(
        _proj_body,
        grid=(n // bn,),
        in_specs=[
            pl.BlockSpec((bn, C * d), lambda i: (i, 0)),
            pl.BlockSpec((C, d, EDGE), lambda i: (0, 0, 0)),
            pl.BlockSpec((C, d, EDGE), lambda i: (0, 0, 0)),
            pl.BlockSpec((C, EDGE), lambda i: (0, 0)),
        ],
        out_specs=[
            pl.BlockSpec((bn, F), lambda i: (i, 0)),
            pl.BlockSpec((bn, F), lambda i: (i, 0)),
        ],
        out_shape=[
            jax.ShapeDtypeStruct((n, F), jnp.float32),
            jax.ShapeDtypeStruct((n, F), jnp.float32),
        ],
    )(xf, wi, wj, eb1)

    xt2 = jnp.pad(jnp.transpose(x, (1, 0, 2)),
                  ((0, 0), (0, 0), (0, DP - d))).reshape(C * n, DP)
    ew2s = jnp.broadcast_to(ew2.reshape(F, 1), (F, 16))
    eb2s = jnp.broadcast_to(eb2.reshape(C, 1), (C, 16))
    zeros_hbm = jnp.zeros((n // 16, DP), jnp.float32)
    nch_tot = e // K
    nch_max = -(-nch_tot // NTILES)
    aggr_parts = _make_sc_call(n, e, nch_max)(
        ai2, aj2, xt2, edge_index, ew2s, eb2s, zeros_hbm)
    aggr_parts = aggr_parts.reshape(2, C, n, DP)

    xtp = xt2.reshape(C, n, DP)
    w1 = jnp.transpose(nw1, (0, 2, 1))
    w2 = jnp.transpose(nw2, (0, 2, 1))
    out = pl.pallas_call(
        _node_body,
        grid=(C, n // bn),
        in_specs=[
            pl.BlockSpec((1, bn, DP), lambda c, i: (c, i, 0)),
            pl.BlockSpec((1, 1, bn, DP), lambda c, i: (0, c, i, 0)),
            pl.BlockSpec((1, 1, bn, DP), lambda c, i: (1, c, i, 0)),
            pl.BlockSpec((1, 2 * d, NODE), lambda c, i: (c, 0, 0)),
            pl.BlockSpec((C, NODE), lambda c, i: (0, 0)),
            pl.BlockSpec((1, NODE, NODE), lambda c, i: (c, 0, 0)),
            pl.BlockSpec((C, NODE), lambda c, i: (0, 0)),
        ],
        out_specs=pl.BlockSpec((1, bn, NODE), lambda c, i: (c, i, 0)),
        out_shape=jax.ShapeDtypeStruct((C, n, NODE), jnp.float32),
    )(xtp, aggr_parts, aggr_parts, w1, nb1, w2, nb2)
    return jnp.transpose(out, (1, 0, 2))


def kernel(x_u, x_v, x_y, edge_index_u, edge_index_v, edge_index_y,
           u_ew1, u_eb1, u_ew2, u_eb2, u_nw1, u_nb1, u_nw2, u_nb2,
           v_ew1, v_eb1, v_ew2, v_eb2, v_nw1, v_nb1, v_nw2, v_nb2,
           y_ew1, y_eb1, y_ew2, y_eb2, y_nw1, y_nb1, y_nw2, y_nb2):
    out_u = _plane(x_u, edge_index_u, u_ew1, u_eb1, u_ew2, u_eb2,
                   u_nw1, u_nb1, u_nw2, u_nb2)
    out_v = _plane(x_v, edge_index_v, v_ew1, v_eb1, v_ew2, v_eb2,
                   v_nw1, v_nb1, v_nw2, v_nb2)
    out_y = _plane(x_y, edge_index_y, y_ew1, y_eb1, y_ew2, y_eb2,
                   y_nw1, y_nb1, y_nw2, y_nb2)
    return (out_u, out_v, out_y)


# speedup vs baseline: 13.6083x; 1.1756x over previous
import functools

import jax
import jax.numpy as jnp
from jax import lax
from jax.experimental import pallas as pl
from jax.experimental.pallas import tpu as pltpu
from jax.experimental.pallas import tpu_sc as plsc

C = 5
EDGE = 16
NODE = 64
F = C * EDGE
DP = 80
K = 128
NTILES = 32


def _proj_body(x_ref, wi_ref, wj_ref, bi_ref, ai_ref, aj_ref):
    x = x_ref[...]
    d = wi_ref.shape[1]
    for c in range(C):
        xc = x[:, c * d:(c + 1) * d]
        ai = jnp.dot(xc, wi_ref[c], preferred_element_type=jnp.float32)
        aj = jnp.dot(xc, wj_ref[c], preferred_element_type=jnp.float32)
        ai_ref[:, c * EDGE:(c + 1) * EDGE] = 2.0 * (ai + bi_ref[c][None, :])
        aj_ref[:, c * EDGE:(c + 1) * EDGE] = 2.0 * aj


def _node_body(x_ref, a0_ref, a1_ref, w1_ref, b1_ref, w2_ref, b2_ref,
               out_ref):
    c = pl.program_id(0)
    d = w1_ref.shape[1] // 2
    xc = x_ref[0, :, :d]
    ac = a0_ref[0, 0, :, :d] + a1_ref[0, 0, :, :d]
    h = jnp.dot(xc, w1_ref[0, :d], preferred_element_type=jnp.float32)
    h = h + jnp.dot(ac, w1_ref[0, d:], preferred_element_type=jnp.float32)
    h = jnp.tanh(h + b1_ref[c][None, :])
    o = jnp.dot(h, w2_ref[0], preferred_element_type=jnp.float32)
    out_ref[0] = jnp.tanh(o + b2_ref[c][None, :])


def _sc_body(ai2, aj2, xt2, eidx, ew2s, eb2s, zeros_hbm, aggr_out,
             srcidx, dstidx, wbuf, ga_a, ga_b,
             srcc_a, srcc_b, ew2v, eb2v, aggr_sh,
             semi, sema0, sema1, semx0, semx1, sems0, sems1):
    n = zeros_hbm.shape[0] * 16
    e = eidx.shape[1]
    nch_tot = e // K
    nper = n // 16

    ci = lax.axis_index("c")
    si = lax.axis_index("s")
    wid = si * 2 + ci
    nch = nch_tot // NTILES + jnp.where(wid < nch_tot % NTILES, 1, 0)

    pltpu.sync_copy(ew2s, ew2v)
    pltpu.sync_copy(eb2s, eb2v)
    iota = lax.iota(jnp.int32, 16)

    def idx_issue(l, c):
        base = (wid + l * NTILES) * K
        pltpu.async_copy(eidx.at[0, pl.ds(base, K)], srcidx.at[l], semi)
        pltpu.async_copy(eidx.at[1, pl.ds(base, K)], dstidx.at[l], semi)
        return c
    lax.fori_loop(0, nch, idx_issue, 0)

    def idx_wait(l, c):
        base = (wid + l * NTILES) * K
        pltpu.make_async_copy(eidx.at[0, pl.ds(base, K)], srcidx.at[l],
                              semi).wait()
        pltpu.make_async_copy(eidx.at[1, pl.ds(base, K)], dstidx.at[l],
                              semi).wait()
        return c
    lax.fori_loop(0, nch, idx_wait, 0)

    def base_issue(l, ga, sem):
        pltpu.async_copy(ai2.at[dstidx.at[l]], ga, sem)

    def base_wait(l, ga, sem):
        pltpu.make_async_copy(ai2.at[dstidx.at[l]], ga, sem).wait()

    def add_issue(l, ga, sem):
        pltpu.async_copy(aj2.at[srcidx.at[l]], ga, sem, add=True)

    def add_wait(l, ga, sem):
        pltpu.make_async_copy(aj2.at[srcidx.at[l]], ga, sem).wait()

    def compute_a(l, ga):
        def grp_body(grp, cc):
            rows = iota + grp * 16
            accs = []
            for c in range(C):
                def fbody(fb, acc, c=c, rows=rows):
                    for ff in range(4):
                        row = c * 16 + fb * 4 + ff
                        colv = jnp.full((16,), row, jnp.int32)
                        z = plsc.load_gather(ga, [rows, colv])
                        ez = jnp.exp(z)
                        t = (ez - 1.0) / (ez + 1.0)
                        acc = acc + ew2v[row, :] * t
                    return acc
                accs.append(lax.fori_loop(0, 4, fbody, eb2v[c, :]))
            m = accs[0]
            for c in range(1, C):
                m = jnp.maximum(m, accs[c])
            es = [jnp.exp(a - m) for a in accs]
            tot = es[0]
            for c in range(1, C):
                tot = tot + es[c]
            r = 1.0 / tot
            for c in range(C):
                wbuf[c, l, pl.ds(grp * 16, 16)] = es[c] * r
            return cc
        lax.fori_loop(0, 8, grp_body, 0)

    base_issue(0, ga_a, sema0)

    def pair_a(i, carry):
        c0 = 2 * i
        c1 = c0 + 1
        base_wait(c0, ga_a, sema0)
        add_issue(c0, ga_a, sema0)
        base_issue(c1, ga_b, sema1)
        add_wait(c0, ga_a, sema0)
        compute_a(c0, ga_a)
        base_wait(c1, ga_b, sema1)
        add_issue(c1, ga_b, sema1)
        @pl.when(c0 + 2 < nch)
        def _():
            base_issue(c0 + 2, ga_a, sema0)
        add_wait(c1, ga_b, sema1)
        compute_a(c1, ga_b)
        return carry
    lax.fori_loop(0, nch // 2, pair_a, 0)

    @pl.when(nch % 2 == 1)
    def _():
        l = nch - 1
        base_wait(l, ga_a, sema0)
        add_issue(l, ga_a, sema0)
        add_wait(l, ga_a, sema0)
        compute_a(l, ga_a)

    def cls_body(cls, carry):
        pltpu.sync_copy(zeros_hbm, aggr_sh.at[pl.ds(si * nper, nper)])
        plsc.subcore_barrier()

        def issue_x(l, srcc, gx, sem):
            def jb(j, c2):
                srcc[pl.ds(j * 16, 16)] = (
                    srcidx[l, pl.ds(j * 16, 16)] + cls * n)
                return c2
            lax.fori_loop(0, 8, jb, 0)
            pltpu.async_copy(xt2.at[srcc], gx, sem)

        def wait_x(srcc, gx, sem):
            pltpu.make_async_copy(xt2.at[srcc], gx, sem).wait()

        def scale_b(l, gx):
            def grp_body(grp, c3):
                rows = iota + grp * 16
                wv = wbuf[cls, l, pl.ds(grp * 16, 16)]
                def fb2(fb, c4, rows=rows, wv=wv):
                    for ff in range(17):
                        colv = jnp.full((16,), fb * 17 + ff, jnp.int32)
                        xc = plsc.load_gather(gx, [rows, colv])
                        plsc.store_scatter(gx, [rows, colv], xc * wv)
                    return c4
                lax.fori_loop(0, 4, fb2, 0)
                return c3
            lax.fori_loop(0, 8, grp_body, 0)

        def issue_s(l, gx, sem):
            pltpu.async_copy(gx, aggr_sh.at[dstidx.at[l]], sem, add=True)

        def wait_s(l, gx, sem):
            pltpu.make_async_copy(gx, aggr_sh.at[dstidx.at[l]], sem).wait()

        issue_x(0, srcc_a, ga_a, semx0)

        def pair_b(i, cc):
            c0 = 2 * i
            c1 = c0 + 1
            @pl.when(i > 0)
            def _():
                wait_s(c0 - 1, ga_b, sems1)
            issue_x(c1, srcc_b, ga_b, semx1)
            wait_x(srcc_a, ga_a, semx0)
            scale_b(c0, ga_a)
            issue_s(c0, ga_a, sems0)
            wait_x(srcc_b, ga_b, semx1)
            scale_b(c1, ga_b)
            issue_s(c1, ga_b, sems1)
            wait_s(c0, ga_a, sems0)
            @pl.when(c0 + 2 < nch)
            def _():
                issue_x(c0 + 2, srcc_a, ga_a, semx0)
            return cc
        lax.fori_loop(0, nch // 2, pair_b, 0)

        @pl.when(nch % 2 == 1)
        def _():
            l = nch - 1
            wait_s(l - 1, ga_b, sems1)
            wait_x(srcc_a, ga_a, semx0)
            scale_b(l, ga_a)
            issue_s(l, ga_a, sems0)
            wait_s(l, ga_a, sems0)

        @pl.when(nch % 2 == 0)
        def _():
            wait_s(nch - 1, ga_b, sems1)

        plsc.subcore_barrier()
        pltpu.sync_copy(aggr_sh.at[pl.ds(si * nper, nper)],
                        aggr_out.at[ci, cls, si])
        plsc.subcore_barrier()
        return carry

    lax.fori_loop(0, C, cls_body, 0)


def _make_sc_call(n, e, nch_max):
    return pl.kernel(
        _sc_body,
        out_type=jax.ShapeDtypeStruct((2, C, 16, n // 16, DP), jnp.float32),
        mesh=plsc.VectorSubcoreMesh(core_axis_name="c", subcore_axis_name="s"),
        compiler_params=pltpu.CompilerParams(
            needs_layout_passes=False, use_tc_tiling_on_sc=False),
        scratch_types=[
            pltpu.VMEM((nch_max, K), jnp.int32),
            pltpu.VMEM((nch_max, K), jnp.int32),
            pltpu.VMEM((C, nch_max, K), jnp.float32),
            pltpu.VMEM((K, F), jnp.float32),
            pltpu.VMEM((K, F), jnp.float32),
            pltpu.VMEM((K,), jnp.int32),
            pltpu.VMEM((K,), jnp.int32),
            pltpu.VMEM((F, 16), jnp.float32),
            pltpu.VMEM((C, 16), jnp.float32),
            pltpu.VMEM_SHARED((n, DP), jnp.float32),
            pltpu.SemaphoreType.DMA,
            pltpu.SemaphoreType.DMA,
            pltpu.SemaphoreType.DMA,
            pltpu.SemaphoreType.DMA,
            pltpu.SemaphoreType.DMA,
            pltpu.SemaphoreType.DMA,
            pltpu.SemaphoreType.DMA,
        ],
    )


def _plane(x, edge_index, ew1, eb1, ew2, eb2, nw1, nb1, nw2, nb2):
    n, _, d = x.shape
    e = edge_index.shape[1]
    xf = x.reshape(n, C * d)

    wi = jnp.transpose(ew1[:, :, :d], (0, 2, 1))
    wj = jnp.transpose(ew1[:, :, d:], (0, 2, 1))
    bn = 1000
    ai2, aj2 = pl.pallas_call(
        _proj_body,
        grid=(n // bn,),
        in_specs=[
            pl.BlockSpec((bn, C * d), lambda i: (i, 0)),
            pl.BlockSpec((C, d, EDGE), lambda i: (0, 0, 0)),
            pl.BlockSpec((C, d, EDGE), lambda i: (0, 0, 0)),
            pl.BlockSpec((C, EDGE), lambda i: (0, 0)),
        ],
        out_specs=[
            pl.BlockSpec((bn, F), lambda i: (i, 0)),
            pl.BlockSpec((bn, F), lambda i: (i, 0)),
        ],
        out_shape=[
            jax.ShapeDtypeStruct((n, F), jnp.float32),
            jax.ShapeDtypeStruct((n, F), jnp.float32),
        ],
    )(xf, wi, wj, eb1)

    xt2 = jnp.pad(jnp.transpose(x, (1, 0, 2)),
                  ((0, 0), (0, 0), (0, DP - d))).reshape(C * n, DP)
    ew2s = jnp.broadcast_to(ew2.reshape(F, 1), (F, 16))
    eb2s = jnp.broadcast_to(eb2.reshape(C, 1), (C, 16))
    zeros_hbm = jnp.zeros((n // 16, DP), jnp.float32)
    nch_tot = e // K
    nch_max = -(-nch_tot // NTILES)
    aggr_parts = _make_sc_call(n, e, nch_max)(
        ai2, aj2, xt2, edge_index, ew2s, eb2s, zeros_hbm)
    aggr_parts = aggr_parts.reshape(2, C, n, DP)

    xtp = xt2.reshape(C, n, DP)
    w1 = jnp.transpose(nw1, (0, 2, 1))
    w2 = jnp.transpose(nw2, (0, 2, 1))
    out = pl.pallas_call(
        _node_body,
        grid=(C, n // bn),
        in_specs=[
            pl.BlockSpec((1, bn, DP), lambda c, i: (c, i, 0)),
            pl.BlockSpec((1, 1, bn, DP), lambda c, i: (0, c, i, 0)),
            pl.BlockSpec((1, 1, bn, DP), lambda c, i: (1, c, i, 0)),
            pl.BlockSpec((1, 2 * d, NODE), lambda c, i: (c, 0, 0)),
            pl.BlockSpec((C, NODE), lambda c, i: (0, 0)),
            pl.BlockSpec((1, NODE, NODE), lambda c, i: (c, 0, 0)),
            pl.BlockSpec((C, NODE), lambda c, i: (0, 0)),
        ],
        out_specs=pl.BlockSpec((1, bn, NODE), lambda c, i: (c, i, 0)),
        out_shape=jax.ShapeDtypeStruct((C, n, NODE), jnp.float32),
    )(xtp, aggr_parts, aggr_parts, w1, nb1, w2, nb2)
    return jnp.transpose(out, (1, 0, 2))


def kernel(x_u, x_v, x_y, edge_index_u, edge_index_v, edge_index_y,
           u_ew1, u_eb1, u_ew2, u_eb2, u_nw1, u_nb1, u_nw2, u_nb2,
           v_ew1, v_eb1, v_ew2, v_eb2, v_nw1, v_nb1, v_nw2, v_nb2,
           y_ew1, y_eb1, y_ew2, y_eb2, y_nw1, y_nb1, y_nw2, y_nb2):
    out_u = _plane(x_u, edge_index_u, u_ew1, u_eb1, u_ew2, u_eb2,
                   u_nw1, u_nb1, u_nw2, u_nb2)
    out_v = _plane(x_v, edge_index_v, v_ew1, v_eb1, v_ew2, v_eb2,
                   v_nw1, v_nb1, v_nw2, v_nb2)
    out_y = _plane(x_y, edge_index_y, y_ew1, y_eb1, y_ew2, y_eb2,
                   y_nw1, y_nb1, y_nw2, y_nb2)
    return (out_u, out_v, out_y)
